# baseline (speedup 1.0000x reference)
import functools

import jax
import jax.numpy as jnp
from jax import lax
from jax.experimental import pallas as pl
from jax.experimental.pallas import tpu as pltpu
from jax.experimental.pallas import tpu_sc as plsc

N_NODES = 10000
N_EDGES = 320000
D = 128

NC = 2
NS = 16
NW = NC * NS
L = 16
K = 80
CH = N_EDGES // (NW * K)
IB = 25
NB = CH // IB
N_PAD = 10112
ROWS_PER_TILE = N_PAD // NS


def _sc_aggregate(x, src3d, dst3d, zagg):
    mesh = plsc.VectorSubcoreMesh(core_axis_name="c", subcore_axis_name="s")

    @functools.partial(
        pl.kernel,
        out_type=(
            jax.ShapeDtypeStruct((NC, N_PAD, D), jnp.float32),
            jax.ShapeDtypeStruct((NW, N_PAD), jnp.float32),
        ),
        mesh=mesh,
        compiler_params=pltpu.CompilerParams(use_tc_tiling_on_sc=False,
                                             needs_layout_passes=False),
        scratch_types=[
            pltpu.VMEM((2, IB, K), jnp.int32),
            pltpu.VMEM((2, IB, K), jnp.int32),
            pltpu.VMEM((K, D), jnp.float32),
            pltpu.VMEM((K, D), jnp.float32),
            pltpu.VMEM((K, D), jnp.float32),
            pltpu.VMEM((N_PAD,), jnp.float32),
            pltpu.SemaphoreType.DMA,
            pltpu.SemaphoreType.DMA,
            pltpu.SemaphoreType.DMA,
            pltpu.SemaphoreType.DMA,
            pltpu.SemaphoreType.DMA,
            pltpu.SemaphoreType.DMA,
            pltpu.SemaphoreType.DMA,
            pltpu.VMEM_SHARED((N_PAD, D), jnp.float32),
        ],
    )
    def sc_body(x_hbm, src_hbm, dst_hbm, zagg_hbm,
                agg_out, deg_out,
                src_v, dst_v, rows_a, rows_b, rows_c, hist_v,
                ga, gb, gc, sa, sb, sc_, sem_i,
                sh_agg):
        c = lax.axis_index("c")
        s = lax.axis_index("s")
        wid = s * NC + c
        r0 = s * ROWS_PER_TILE
        pltpu.sync_copy(zagg_hbm.at[pl.ds(r0, ROWS_PER_TILE)],
                        sh_agg.at[pl.ds(r0, ROWS_PER_TILE)])
        zeros16 = jnp.zeros((L,), jnp.float32)

        def zbody(i, carry):
            hist_v[pl.ds(i * L, L)] = zeros16
            return carry

        lax.fori_loop(0, N_PAD // L, zbody, 0)
        pltpu.async_copy(src_hbm.at[wid, 0], src_v.at[0], sem_i)
        pltpu.async_copy(dst_hbm.at[wid, 0], dst_v.at[0], sem_i)
        plsc.subcore_barrier()

        ones16 = jnp.ones((L,), jnp.float32)

        def histo(pb, j):
            for i in range(K // L):
                idx = dst_v[pb, j, pl.ds(i * L, L)]
                plsc.addupdate_scatter(hist_v, [idx], ones16)

        def blk_body(bk, carry):
            pb = bk % 2
            pltpu.make_async_copy(src_hbm.at[wid, bk], src_v.at[pb], sem_i).wait()
            pltpu.make_async_copy(dst_hbm.at[wid, bk], dst_v.at[pb], sem_i).wait()

            @pl.when(bk + 1 < NB)
            def _():
                pltpu.async_copy(src_hbm.at[wid, bk + 1], src_v.at[1 - pb], sem_i)
                pltpu.async_copy(dst_hbm.at[wid, bk + 1], dst_v.at[1 - pb], sem_i)

            def g_start(j, rows, sem):
                pltpu.async_copy(x_hbm.at[src_v.at[pb, j]], rows, sem)

            def g_wait(rows, sem):
                pltpu.make_async_copy(x_hbm.at[src_v.at[pb, 0]], rows, sem).wait()

            def s_start(j, rows, sem):
                pltpu.async_copy(rows, sh_agg.at[dst_v.at[pb, j]], sem, add=True)

            def s_wait(rows, sem):
                pltpu.make_async_copy(rows, sh_agg.at[dst_v.at[pb, 0]], sem).wait()

            g_start(0, rows_a, ga)
            g_start(1, rows_b, gb)
            g_start(2, rows_c, gc)

            def body(t, c2):
                j = 3 * t
                histo(pb, j)
                g_wait(rows_a, ga)
                s_start(j, rows_a, sa)
                histo(pb, j + 1)
                g_wait(rows_b, gb)
                s_start(j + 1, rows_b, sb)
                histo(pb, j + 2)
                g_wait(rows_c, gc)
                s_start(j + 2, rows_c, sc_)
                s_wait(rows_a, sa)
                g_start(j + 3, rows_a, ga)
                s_wait(rows_b, sb)
                g_start(j + 4, rows_b, gb)
                s_wait(rows_c, sc_)
                g_start(j + 5, rows_c, gc)
                return c2

            lax.fori_loop(0, (IB - 4) // 3, body, 0)
            histo(pb, IB - 4)
            g_wait(rows_a, ga)
            s_start(IB - 4, rows_a, sa)
            histo(pb, IB - 3)
            g_wait(rows_b, gb)
            s_start(IB - 3, rows_b, sb)
            histo(pb, IB - 2)
            g_wait(rows_c, gc)
            s_start(IB - 2, rows_c, sc_)
            s_wait(rows_a, sa)
            g_start(IB - 1, rows_a, ga)
            histo(pb, IB - 1)
            g_wait(rows_a, ga)
            s_start(IB - 1, rows_a, sa)
            s_wait(rows_a, sa)
            s_wait(rows_b, sb)
            s_wait(rows_c, sc_)
            return carry

        lax.fori_loop(0, NB, blk_body, 0)
        pltpu.sync_copy(hist_v, deg_out.at[wid])
        plsc.subcore_barrier()
        pltpu.sync_copy(sh_agg.at[pl.ds(r0, ROWS_PER_TILE)],
                        agg_out.at[c, pl.ds(r0, ROWS_PER_TILE)])

    return sc_body(x, src3d, dst3d, zagg)


def _tc_body(x_r, agg_r, deg_r, ws_r, wn_r, b_r, o_r):
    a = agg_r[0] + agg_r[1]
    d = jnp.sum(deg_r[...], axis=0)[:, None]
    a = a / jnp.maximum(d, 1.0)
    o = (jnp.dot(x_r[...], ws_r[...], preferred_element_type=jnp.float32)
         + jnp.dot(a, wn_r[...], preferred_element_type=jnp.float32)
         + b_r[...])
    o_r[...] = jnp.maximum(o, 0.0)


def _tc_combine(x, agg_p, deg_p, W_self, W_neigh, b2):
    B = 1280
    grid = (pl.cdiv(N_NODES, B),)
    return pl.pallas_call(
        _tc_body,
        grid=grid,
        in_specs=[
            pl.BlockSpec((B, D), lambda i: (i, 0)),
            pl.BlockSpec((NC, B, D), lambda i: (0, i, 0)),
            pl.BlockSpec((NW, B), lambda i: (0, i)),
            pl.BlockSpec((D, D), lambda i: (0, 0)),
            pl.BlockSpec((D, D), lambda i: (0, 0)),
            pl.BlockSpec((1, D), lambda i: (0, 0)),
        ],
        out_specs=pl.BlockSpec((B, D), lambda i: (i, 0)),
        out_shape=jax.ShapeDtypeStruct((N_NODES, D), jnp.float32),
    )(x, agg_p, deg_p, W_self, W_neigh, b2)


def kernel(x, edge_index, W_self, W_neigh, b):
    src3d = edge_index[0].reshape(NW, NB, IB, K)
    dst3d = edge_index[1].reshape(NW, NB, IB, K)
    zagg = jnp.zeros((N_PAD, D), jnp.float32)
    agg_p, deg_p = _sc_aggregate(x, src3d, dst3d, zagg)
    return _tc_combine(x, agg_p, deg_p, W_self, W_neigh, b.reshape(1, D))


# speedup vs baseline: 1.0280x; 1.0280x over previous
import functools

import jax
import jax.numpy as jnp
from jax import lax
from jax.experimental import pallas as pl
from jax.experimental.pallas import tpu as pltpu
from jax.experimental.pallas import tpu_sc as plsc

N_NODES = 10000
N_EDGES = 320000
D = 128

NC = 2
NS = 16
NW = NC * NS
L = 16
K = 80
CH = N_EDGES // (NW * K)
IB = 25
NB = CH // IB
N_PAD = 10240
ROWS_PER_TILE = N_PAD // NS


def _sc_aggregate(x, src3d, dst3d, zagg):
    mesh = plsc.VectorSubcoreMesh(core_axis_name="c", subcore_axis_name="s")

    @functools.partial(
        pl.kernel,
        out_type=(
            jax.ShapeDtypeStruct((NC, N_PAD, D), jnp.float32),
            jax.ShapeDtypeStruct((NW, N_PAD), jnp.float32),
        ),
        mesh=mesh,
        compiler_params=pltpu.CompilerParams(use_tc_tiling_on_sc=False,
                                             needs_layout_passes=False),
        scratch_types=[
            pltpu.VMEM((2, IB, K), jnp.int32),
            pltpu.VMEM((2, IB, K), jnp.int32),
            pltpu.VMEM((K, D), jnp.float32),
            pltpu.VMEM((K, D), jnp.float32),
            pltpu.VMEM((N_PAD,), jnp.float32),
            pltpu.SemaphoreType.DMA,
            pltpu.SemaphoreType.DMA,
            pltpu.SemaphoreType.DMA,
            pltpu.VMEM_SHARED((N_PAD, D), jnp.float32),
        ],
    )
    def sc_body(x_hbm, src_hbm, dst_hbm, zagg_hbm,
                agg_out, deg_out,
                src_v, dst_v, rows_a, rows_b, hist_v, sem_a, sem_b, sem_i,
                sh_agg):
        c = lax.axis_index("c")
        s = lax.axis_index("s")
        wid = s * NC + c
        r0 = s * ROWS_PER_TILE
        pltpu.sync_copy(zagg_hbm.at[pl.ds(r0, ROWS_PER_TILE)],
                        sh_agg.at[pl.ds(r0, ROWS_PER_TILE)])
        zeros16 = jnp.zeros((L,), jnp.float32)

        def zbody(i, carry):
            hist_v[pl.ds(i * L, L)] = zeros16
            return carry

        lax.fori_loop(0, N_PAD // L, zbody, 0)
        pltpu.async_copy(src_hbm.at[wid, 0], src_v.at[0], sem_i)
        pltpu.async_copy(dst_hbm.at[wid, 0], dst_v.at[0], sem_i)
        plsc.subcore_barrier()

        ones16 = jnp.ones((L,), jnp.float32)

        def histo(dref, j):
            for i in range(K // L):
                idx = dref[j, pl.ds(i * L, L)]
                plsc.addupdate_scatter(hist_v, [idx], ones16)

        def blk_body(bk, carry):
            sb = bk % 2
            pltpu.make_async_copy(src_hbm.at[wid, bk], src_v.at[sb], sem_i).wait()
            pltpu.make_async_copy(dst_hbm.at[wid, bk], dst_v.at[sb], sem_i).wait()
            src_b, dst_b = src_v.at[sb], dst_v.at[sb]

            @pl.when(bk + 1 < NB)
            def _():
                pltpu.async_copy(src_hbm.at[wid, bk + 1], src_v.at[1 - sb], sem_i)
                pltpu.async_copy(dst_hbm.at[wid, bk + 1], dst_v.at[1 - sb], sem_i)

            pltpu.async_copy(x_hbm.at[src_b.at[0]], rows_a, sem_a)

            def body(t, c2):
                j = 2 * t
                pltpu.async_copy(x_hbm.at[src_b.at[j + 1]], rows_b, sem_b)
                histo(dst_b, j)
                pltpu.make_async_copy(x_hbm.at[src_b.at[j]], rows_a, sem_a).wait()
                pltpu.sync_copy(rows_a, sh_agg.at[dst_b.at[j]], add=True)
                pltpu.async_copy(x_hbm.at[src_b.at[j + 2]], rows_a, sem_a)
                histo(dst_b, j + 1)
                pltpu.make_async_copy(x_hbm.at[src_b.at[j + 1]], rows_b, sem_b).wait()
                pltpu.sync_copy(rows_b, sh_agg.at[dst_b.at[j + 1]], add=True)
                return c2

            lax.fori_loop(0, (IB - 1) // 2, body, 0)
            histo(dst_b, IB - 1)
            pltpu.make_async_copy(x_hbm.at[src_b.at[IB - 1]], rows_a, sem_a).wait()
            pltpu.sync_copy(rows_a, sh_agg.at[dst_b.at[IB - 1]], add=True)
            return carry

        lax.fori_loop(0, NB, blk_body, 0)
        pltpu.sync_copy(hist_v, deg_out.at[wid])
        plsc.subcore_barrier()
        pltpu.sync_copy(sh_agg.at[pl.ds(r0, ROWS_PER_TILE)],
                        agg_out.at[c, pl.ds(r0, ROWS_PER_TILE)])

    return sc_body(x, src3d, dst3d, zagg)


def _tc_body(x_r, agg_r, deg_r, ws_r, wn_r, b_r, o_r):
    a = agg_r[0, :N_NODES] + agg_r[1, :N_NODES]
    d = jnp.sum(deg_r[...], axis=0)[:N_NODES, None]
    a = a / jnp.maximum(d, 1.0)
    o = (jnp.dot(x_r[...], ws_r[...], preferred_element_type=jnp.float32)
         + jnp.dot(a, wn_r[...], preferred_element_type=jnp.float32)
         + b_r[...])
    o_r[...] = jnp.maximum(o, 0.0)


def _tc_combine(x, agg_p, deg_p, W_self, W_neigh, b2):
    return pl.pallas_call(
        _tc_body,
        out_shape=jax.ShapeDtypeStruct((N_NODES, D), jnp.float32),
    )(x, agg_p, deg_p, W_self, W_neigh, b2)


def kernel(x, edge_index, W_self, W_neigh, b):
    src3d = edge_index[0].reshape(NW, NB, IB, K)
    dst3d = edge_index[1].reshape(NW, NB, IB, K)
    zagg = jnp.zeros((N_PAD, D), jnp.float32)
    agg_p, deg_p = _sc_aggregate(x, src3d, dst3d, zagg)
    return _tc_combine(x, agg_p, deg_p, W_self, W_neigh, b.reshape(1, D))


# speedup vs baseline: 1.0815x; 1.0520x over previous
import functools

import jax
import jax.numpy as jnp
from jax import lax
from jax.experimental import pallas as pl
from jax.experimental.pallas import tpu as pltpu
from jax.experimental.pallas import tpu_sc as plsc

N_NODES = 10000
N_EDGES = 320000
D = 128

NC = 2
NS = 16
NW = NC * NS
L = 16
K = 80
CH = N_EDGES // (NW * K)
IB = 25
NB = CH // IB
N_PAD = 10240
ROWS_PER_TILE = N_PAD // NS


def _sc_aggregate(x, src3d, dst3d, zagg):
    mesh = plsc.VectorSubcoreMesh(core_axis_name="c", subcore_axis_name="s")

    @functools.partial(
        pl.kernel,
        out_type=(
            jax.ShapeDtypeStruct((NC, N_PAD, D), jnp.bfloat16),
            jax.ShapeDtypeStruct((NW, N_PAD), jnp.float32),
        ),
        mesh=mesh,
        compiler_params=pltpu.CompilerParams(use_tc_tiling_on_sc=False,
                                             needs_layout_passes=False),
        scratch_types=[
            pltpu.VMEM((2, IB, K), jnp.int32),
            pltpu.VMEM((2, IB, K), jnp.int32),
            pltpu.VMEM((K, D), jnp.bfloat16),
            pltpu.VMEM((K, D), jnp.bfloat16),
            pltpu.VMEM((N_PAD,), jnp.float32),
            pltpu.SemaphoreType.DMA,
            pltpu.SemaphoreType.DMA,
            pltpu.SemaphoreType.DMA,
            pltpu.VMEM_SHARED((N_PAD, D), jnp.bfloat16),
        ],
    )
    def sc_body(x_hbm, src_hbm, dst_hbm, zagg_hbm,
                agg_out, deg_out,
                src_v, dst_v, rows_a, rows_b, hist_v, sem_a, sem_b, sem_i,
                sh_agg):
        c = lax.axis_index("c")
        s = lax.axis_index("s")
        wid = s * NC + c
        r0 = s * ROWS_PER_TILE
        pltpu.sync_copy(zagg_hbm.at[pl.ds(r0, ROWS_PER_TILE)],
                        sh_agg.at[pl.ds(r0, ROWS_PER_TILE)])
        zeros16 = jnp.zeros((L,), jnp.float32)

        def zbody(i, carry):
            hist_v[pl.ds(i * L, L)] = zeros16
            return carry

        lax.fori_loop(0, N_PAD // L, zbody, 0)
        pltpu.async_copy(src_hbm.at[wid, 0], src_v.at[0], sem_i)
        pltpu.async_copy(dst_hbm.at[wid, 0], dst_v.at[0], sem_i)
        plsc.subcore_barrier()

        ones16 = jnp.ones((L,), jnp.float32)

        def histo(dref, j):
            for i in range(K // L):
                idx = dref[j, pl.ds(i * L, L)]
                plsc.addupdate_scatter(hist_v, [idx], ones16)

        def blk_body(bk, carry):
            sb = bk % 2
            pltpu.make_async_copy(src_hbm.at[wid, bk], src_v.at[sb], sem_i).wait()
            pltpu.make_async_copy(dst_hbm.at[wid, bk], dst_v.at[sb], sem_i).wait()
            src_b, dst_b = src_v.at[sb], dst_v.at[sb]

            @pl.when(bk + 1 < NB)
            def _():
                pltpu.async_copy(src_hbm.at[wid, bk + 1], src_v.at[1 - sb], sem_i)
                pltpu.async_copy(dst_hbm.at[wid, bk + 1], dst_v.at[1 - sb], sem_i)

            pltpu.async_copy(x_hbm.at[src_b.at[0]], rows_a, sem_a)

            def body(t, c2):
                j = 2 * t
                pltpu.async_copy(x_hbm.at[src_b.at[j + 1]], rows_b, sem_b)
                histo(dst_b, j)
                pltpu.make_async_copy(x_hbm.at[src_b.at[j]], rows_a, sem_a).wait()
                pltpu.sync_copy(rows_a, sh_agg.at[dst_b.at[j]], add=True)
                pltpu.async_copy(x_hbm.at[src_b.at[j + 2]], rows_a, sem_a)
                histo(dst_b, j + 1)
                pltpu.make_async_copy(x_hbm.at[src_b.at[j + 1]], rows_b, sem_b).wait()
                pltpu.sync_copy(rows_b, sh_agg.at[dst_b.at[j + 1]], add=True)
                return c2

            lax.fori_loop(0, (IB - 1) // 2, body, 0)
            histo(dst_b, IB - 1)
            pltpu.make_async_copy(x_hbm.at[src_b.at[IB - 1]], rows_a, sem_a).wait()
            pltpu.sync_copy(rows_a, sh_agg.at[dst_b.at[IB - 1]], add=True)
            return carry

        lax.fori_loop(0, NB, blk_body, 0)
        pltpu.sync_copy(hist_v, deg_out.at[wid])
        plsc.subcore_barrier()
        pltpu.sync_copy(sh_agg.at[pl.ds(r0, ROWS_PER_TILE)],
                        agg_out.at[c, pl.ds(r0, ROWS_PER_TILE)])

    return sc_body(x, src3d, dst3d, zagg)


def _tc_body(x_r, agg_r, deg_r, ws_r, wn_r, b_r, o_r):
    a = (agg_r[0, :N_NODES].astype(jnp.float32)
         + agg_r[1, :N_NODES].astype(jnp.float32))
    d = jnp.sum(deg_r[...], axis=0)[:N_NODES, None]
    a = a / jnp.maximum(d, 1.0)
    o = (jnp.dot(x_r[...], ws_r[...], preferred_element_type=jnp.float32)
         + jnp.dot(a, wn_r[...], preferred_element_type=jnp.float32)
         + b_r[...])
    o_r[...] = jnp.maximum(o, 0.0)


def _tc_combine(x, agg_p, deg_p, W_self, W_neigh, b2):
    return pl.pallas_call(
        _tc_body,
        out_shape=jax.ShapeDtypeStruct((N_NODES, D), jnp.float32),
    )(x, agg_p, deg_p, W_self, W_neigh, b2)


def kernel(x, edge_index, W_self, W_neigh, b):
    src3d = edge_index[0].reshape(NW, NB, IB, K)
    dst3d = edge_index[1].reshape(NW, NB, IB, K)
    zagg = jnp.zeros((N_PAD, D), jnp.bfloat16)
    agg_p, deg_p = _sc_aggregate(x.astype(jnp.bfloat16), src3d, dst3d, zagg)
    return _tc_combine(x, agg_p, deg_p, W_self, W_neigh, b.reshape(1, D))
